# baseline (speedup 1.0000x reference)
import functools

import jax
import jax.numpy as jnp
from jax import lax
from jax.experimental import pallas as pl
from jax.experimental.pallas import tpu as pltpu
from jax.experimental.pallas import tpu_sc as plsc

N = 10000
E = 160000
D = 256
DH = D // 2

NC = 2
NS = 16
CHUNK = 128

N_PAD = 10240
ROWS_PER_TILE = N_PAD // NS
ROW_CHUNKS = ROWS_PER_TILE // CHUNK

E_PAD = 163840
NCHUNKS = E_PAD // CHUNK
CHUNKS_PER_TILE = NCHUNKS // NS
PAIRS = CHUNKS_PER_TILE // 2
DEG_CHUNKS_PER_TILE = NCHUNKS // (NC * NS)

R = 1024
GRID = N_PAD // R

_mesh = plsc.VectorSubcoreMesh(core_axis_name="c", subcore_axis_name="s")


@functools.partial(
    pl.kernel,
    out_type=jax.ShapeDtypeStruct((NC * NS, N_PAD), jnp.float32),
    mesh=_mesh,
    scratch_types=[
        pltpu.VMEM((DEG_CHUNKS_PER_TILE, CHUNK), jnp.int32),
        pltpu.VMEM((N_PAD,), jnp.float32),
    ],
    compiler_params=pltpu.CompilerParams(needs_layout_passes=False),
)
def _sc_degree(dst_hbm, out_hbm, idx_v, hist_v):
    c = lax.axis_index("c")
    s = lax.axis_index("s")
    wid = c * NS + s

    pltpu.sync_copy(dst_hbm.at[pl.ds(wid * DEG_CHUNKS_PER_TILE, DEG_CHUNKS_PER_TILE)], idx_v)

    zeros16 = jnp.zeros((16,), jnp.float32)

    def zbody(i, carry):
        for v in range(8):
            hist_v[pl.ds((i * 8 + v) * 16, 16)] = zeros16
        return carry

    lax.fori_loop(0, N_PAD // 128, zbody, 0)

    ones16 = jnp.full((16,), 1.0, jnp.float32)

    def body(j, carry):
        for v in range(8):
            idx = idx_v[j, pl.ds(v * 16, 16)]
            plsc.addupdate_scatter(hist_v, [idx], ones16)
        return carry

    lax.fori_loop(0, DEG_CHUNKS_PER_TILE, body, 0)

    pltpu.sync_copy(hist_v, out_hbm.at[wid])


GC = 16
GROUPS = CHUNKS_PER_TILE // GC
GPAIRS = GC // 2


@functools.partial(
    pl.kernel,
    out_type=jax.ShapeDtypeStruct((NC * N_PAD, DH), jnp.float32),
    mesh=_mesh,
    scratch_types=[
        pltpu.VMEM((GC, CHUNK), jnp.int32),
        pltpu.VMEM((GC, CHUNK), jnp.int32),
        pltpu.VMEM((2, CHUNK, DH), jnp.float32),
        pltpu.VMEM_SHARED((N_PAD, DH), jnp.float32),
        pltpu.SemaphoreType.DMA,
        pltpu.SemaphoreType.DMA,
    ],
    compiler_params=pltpu.CompilerParams(needs_layout_passes=False),
)
def _sc_propagate(p_hbm, src_hbm, dst_hbm, out_hbm,
                  src_v, dst_v, gbuf, acc, sem0, sem1):
    c = lax.axis_index("c")
    s = lax.axis_index("s")
    base_chunk = s * CHUNKS_PER_TILE

    zeros16 = jnp.zeros((16,), jnp.float32)

    def zbody(i, carry):
        for v in range(8):
            gbuf[0, i, pl.ds(v * 16, 16)] = zeros16
        return carry

    lax.fori_loop(0, CHUNK, zbody, 0)
    for k in range(ROW_CHUNKS):
        pltpu.sync_copy(gbuf.at[0], acc.at[pl.ds(s * ROWS_PER_TILE + k * CHUNK, CHUNK)])
    plsc.subcore_barrier()

    for g in range(GROUPS):
        pltpu.sync_copy(src_hbm.at[c, pl.ds(base_chunk + g * GC, GC)], src_v)
        pltpu.sync_copy(dst_hbm.at[pl.ds(base_chunk + g * GC, GC)], dst_v)
        pltpu.async_copy(p_hbm.at[src_v.at[0]], gbuf.at[0], sem0)

        def body(jj, carry):
            j0 = 2 * jj
            j1 = j0 + 1
            pltpu.async_copy(p_hbm.at[src_v.at[j1]], gbuf.at[1], sem1)
            pltpu.make_async_copy(p_hbm.at[src_v.at[j0]], gbuf.at[0], sem0).wait()
            pltpu.sync_copy(gbuf.at[0], acc.at[dst_v.at[j0]], add=True)

            @pl.when(jj < GPAIRS - 1)
            def _():
                pltpu.async_copy(p_hbm.at[src_v.at[j0 + 2]], gbuf.at[0], sem0)

            pltpu.make_async_copy(p_hbm.at[src_v.at[j1]], gbuf.at[1], sem1).wait()
            pltpu.sync_copy(gbuf.at[1], acc.at[dst_v.at[j1]], add=True)
            return carry

        lax.fori_loop(0, GPAIRS, body, 0)
    plsc.subcore_barrier()

    for k in range(ROW_CHUNKS):
        rows = s * ROWS_PER_TILE + k * CHUNK
        pltpu.sync_copy(acc.at[pl.ds(rows, CHUNK)], gbuf.at[0])
        pltpu.sync_copy(gbuf.at[0], out_hbm.at[pl.ds(c * N_PAD + rows, CHUNK)])


def _dinv_of(hist_blk):
    deg = jnp.sum(hist_blk, axis=0) + 1.0
    return 1.0 / jnp.sqrt(deg)


def _ln(z, g, b):
    mu = jnp.mean(z, axis=-1, keepdims=True)
    var = jnp.mean((z - mu) ** 2, axis=-1, keepdims=True)
    return (z - mu) / jnp.sqrt(var + 1e-5) * g + b


def _tc1_body(hist_ref, x_ref, g_ref, b_ref, out_ref):
    dinv = _dinv_of(hist_ref[...])
    p = _ln(x_ref[...], g_ref[...], b_ref[...]) * dinv[:, None]
    out_ref[...] = jnp.stack([p[:, :DH], p[:, DH:]], axis=0)


def _tc2_body(hist_ref, s_ref, p_ref, W_ref, b_ref, g2_ref, be2_ref, out_ref):
    dinv = _dinv_of(hist_ref[...])
    sv = s_ref[...]
    pv = p_ref[...]
    t = jnp.concatenate([sv[0] + pv[0], sv[1] + pv[1]], axis=1) * dinv[:, None]
    h = jnp.dot(t, W_ref[...], preferred_element_type=jnp.float32) + b_ref[...]
    p2 = _ln(h, g2_ref[...], be2_ref[...]) * dinv[:, None]
    out_ref[...] = jnp.stack([p2[:, :DH], p2[:, DH:]], axis=0)


def _tc3_body(hist_ref, s_ref, p_ref, W_ref, b_ref, out_ref):
    dinv = _dinv_of(hist_ref[...])
    sv = s_ref[...]
    pv = p_ref[...]
    t = jnp.concatenate([sv[0] + pv[0], sv[1] + pv[1]], axis=1) * dinv[:, None]
    out_ref[...] = jnp.dot(t, W_ref[...], preferred_element_type=jnp.float32) + b_ref[...]


_hist_spec = pl.BlockSpec((NC * NS, R), lambda i: (0, i))
_row_spec = pl.BlockSpec((R, D), lambda i: (i, 0))
_half_spec = pl.BlockSpec((2, R, DH), lambda i: (0, i, 0))
_vec_spec = pl.BlockSpec((1, D), lambda i: (0, 0))
_mat_spec = pl.BlockSpec((D, D), lambda i: (0, 0))

_p_shape = jax.ShapeDtypeStruct((2, N_PAD, DH), jnp.float32)

_tc1 = pl.pallas_call(
    _tc1_body, grid=(GRID,),
    in_specs=[_hist_spec, _row_spec, _vec_spec, _vec_spec],
    out_specs=_half_spec, out_shape=_p_shape)

_tc2 = pl.pallas_call(
    _tc2_body, grid=(GRID,),
    in_specs=[_hist_spec, _half_spec, _half_spec, _mat_spec, _vec_spec,
              _vec_spec, _vec_spec],
    out_specs=_half_spec, out_shape=_p_shape)

_tc3 = pl.pallas_call(
    _tc3_body, grid=(GRID,),
    in_specs=[_hist_spec, _half_spec, _half_spec, _mat_spec, _vec_spec],
    out_specs=_row_spec,
    out_shape=jax.ShapeDtypeStruct((N_PAD, D), jnp.float32))


@jax.jit
def _run(x, edge_index, ln1_gamma, ln1_beta, W1, b1, ln2_gamma, ln2_beta, W2, b2):
    src = edge_index[0].astype(jnp.int32)
    dst = edge_index[1].astype(jnp.int32)
    pad = E_PAD - E
    src_p = jnp.concatenate([src, jnp.zeros((pad,), jnp.int32)]).reshape(NCHUNKS, CHUNK)
    dst_p = jnp.concatenate(
        [dst, jnp.full((pad,), N_PAD - 1, jnp.int32)]).reshape(NCHUNKS, CHUNK)
    src3 = jnp.stack([src_p, src_p + N_PAD], axis=0)

    x_p = jnp.pad(x, ((0, N_PAD - N), (0, 0)))
    g1 = ln1_gamma.reshape(1, D)
    be1 = ln1_beta.reshape(1, D)
    g2 = ln2_gamma.reshape(1, D)
    be2 = ln2_beta.reshape(1, D)
    b1r = b1.reshape(1, D)
    b2r = b2.reshape(1, D)

    hist = _sc_degree(dst_p)

    p1 = _tc1(hist, x_p, g1, be1)
    s1 = _sc_propagate(p1.reshape(NC * N_PAD, DH), src3, dst_p)
    p2 = _tc2(hist, s1.reshape(2, N_PAD, DH), p1, W1, b1r, g2, be2)
    s2 = _sc_propagate(p2.reshape(NC * N_PAD, DH), src3, dst_p)
    out = _tc3(hist, s2.reshape(2, N_PAD, DH), p2, W2, b2r)
    return out[:N]


def kernel(x, edge_index, ln1_gamma, ln1_beta, W1, b1, ln2_gamma, ln2_beta, W2, b2):
    return _run(x, edge_index, ln1_gamma, ln1_beta, W1, b1,
                ln2_gamma, ln2_beta, W2, b2)


# speedup vs baseline: 2.2221x; 2.2221x over previous
import functools

import jax
import jax.numpy as jnp
from jax import lax
from jax.experimental import pallas as pl
from jax.experimental.pallas import tpu as pltpu
from jax.experimental.pallas import tpu_sc as plsc

N = 10000
E = 160000
D = 256
DH = D // 2

NC = 2
NS = 16
CHUNK = 128

N_PAD = 10240
ROWS_PER_TILE = N_PAD // NS
ROW_CHUNKS = ROWS_PER_TILE // CHUNK

E_PAD = 163840
NCHUNKS = E_PAD // CHUNK
CHUNKS_PER_TILE = NCHUNKS // NS
DEG_CHUNKS_PER_TILE = NCHUNKS // (NC * NS)

R = 1024
GRID = N_PAD // R

_mesh = plsc.VectorSubcoreMesh(core_axis_name="c", subcore_axis_name="s")


@functools.partial(
    pl.kernel,
    out_type=jax.ShapeDtypeStruct((NC * NS, N_PAD), jnp.float32),
    mesh=_mesh,
    scratch_types=[
        pltpu.VMEM((DEG_CHUNKS_PER_TILE, CHUNK), jnp.int32),
        pltpu.VMEM((N_PAD,), jnp.float32),
    ],
    compiler_params=pltpu.CompilerParams(needs_layout_passes=False),
)
def _sc_degree(dst_hbm, out_hbm, idx_v, hist_v):
    c = lax.axis_index("c")
    s = lax.axis_index("s")
    wid = c * NS + s

    pltpu.sync_copy(dst_hbm.at[pl.ds(wid * DEG_CHUNKS_PER_TILE, DEG_CHUNKS_PER_TILE)], idx_v)

    zeros16 = jnp.zeros((16,), jnp.float32)

    def zbody(i, carry):
        for v in range(8):
            hist_v[pl.ds((i * 8 + v) * 16, 16)] = zeros16
        return carry

    lax.fori_loop(0, N_PAD // 128, zbody, 0)

    ones16 = jnp.full((16,), 1.0, jnp.float32)

    def body(j, carry):
        for v in range(8):
            idx = idx_v[j, pl.ds(v * 16, 16)]
            plsc.addupdate_scatter(hist_v, [idx], ones16)
        return carry

    lax.fori_loop(0, DEG_CHUNKS_PER_TILE, body, 0)

    pltpu.sync_copy(hist_v, out_hbm.at[wid])


GC = 16
GROUPS = CHUNKS_PER_TILE // GC
GPAIRS = GC // 2


@functools.partial(
    pl.kernel,
    out_type=jax.ShapeDtypeStruct((NC * N_PAD, DH), jnp.float32),
    mesh=_mesh,
    scratch_types=[
        pltpu.VMEM((GC, CHUNK), jnp.int32),
        pltpu.VMEM((GC, CHUNK), jnp.int32),
        pltpu.VMEM((2, CHUNK, DH), jnp.float32),
        pltpu.VMEM_SHARED((N_PAD, DH), jnp.float32),
        pltpu.SemaphoreType.DMA,
        pltpu.SemaphoreType.DMA,
    ],
    compiler_params=pltpu.CompilerParams(needs_layout_passes=False),
)
def _sc_propagate(p_hbm, src_hbm, dst_hbm, out_hbm,
                  src_v, dst_v, gbuf, acc, sem0, sem1):
    c = lax.axis_index("c")
    s = lax.axis_index("s")
    base_chunk = s * CHUNKS_PER_TILE

    zeros16 = jnp.zeros((16,), jnp.float32)

    def zbody(i, carry):
        for v in range(8):
            gbuf[0, i, pl.ds(v * 16, 16)] = zeros16
        return carry

    lax.fori_loop(0, CHUNK, zbody, 0)
    for k in range(ROW_CHUNKS):
        pltpu.sync_copy(gbuf.at[0], acc.at[pl.ds(s * ROWS_PER_TILE + k * CHUNK, CHUNK)])
    plsc.subcore_barrier()

    for g in range(GROUPS):
        pltpu.sync_copy(src_hbm.at[c, pl.ds(base_chunk + g * GC, GC)], src_v)
        pltpu.sync_copy(dst_hbm.at[pl.ds(base_chunk + g * GC, GC)], dst_v)
        pltpu.async_copy(p_hbm.at[src_v.at[0]], gbuf.at[0], sem0)

        def body(jj, carry):
            j0 = 2 * jj
            j1 = j0 + 1
            pltpu.async_copy(p_hbm.at[src_v.at[j1]], gbuf.at[1], sem1)
            pltpu.make_async_copy(p_hbm.at[src_v.at[j0]], gbuf.at[0], sem0).wait()
            pltpu.sync_copy(gbuf.at[0], acc.at[dst_v.at[j0]], add=True)

            @pl.when(jj < GPAIRS - 1)
            def _():
                pltpu.async_copy(p_hbm.at[src_v.at[j0 + 2]], gbuf.at[0], sem0)

            pltpu.make_async_copy(p_hbm.at[src_v.at[j1]], gbuf.at[1], sem1).wait()
            pltpu.sync_copy(gbuf.at[1], acc.at[dst_v.at[j1]], add=True)
            return carry

        lax.fori_loop(0, GPAIRS, body, 0)
    plsc.subcore_barrier()

    for k in range(ROW_CHUNKS):
        rows = s * ROWS_PER_TILE + k * CHUNK
        pltpu.sync_copy(acc.at[pl.ds(rows, CHUNK)], gbuf.at[0])
        pltpu.sync_copy(gbuf.at[0], out_hbm.at[pl.ds(c * N_PAD + rows, CHUNK)])


def _dinv_of(hist_blk):
    deg = jnp.sum(hist_blk, axis=0) + 1.0
    return 1.0 / jnp.sqrt(deg)


def _ln(z, g, b):
    mu = jnp.mean(z, axis=-1, keepdims=True)
    var = jnp.mean((z - mu) ** 2, axis=-1, keepdims=True)
    return (z - mu) / jnp.sqrt(var + 1e-5) * g + b


def _tc1_body(hist_ref, x_ref, g_ref, b_ref, out_ref):
    dinv = _dinv_of(hist_ref[...])
    p = _ln(x_ref[...], g_ref[...], b_ref[...]) * dinv[:, None]
    out_ref[...] = jnp.stack([p[:, :DH], p[:, DH:]], axis=0)


def _tc2_body(hist_ref, s_ref, p_ref, W_ref, b_ref, g2_ref, be2_ref, out_ref):
    dinv = _dinv_of(hist_ref[...])
    sv = s_ref[...]
    pv = p_ref[...]
    t = jnp.concatenate([sv[0] + pv[0], sv[1] + pv[1]], axis=1) * dinv[:, None]
    h = jnp.dot(t, W_ref[...], preferred_element_type=jnp.float32) + b_ref[...]
    p2 = _ln(h, g2_ref[...], be2_ref[...]) * dinv[:, None]
    out_ref[...] = jnp.stack([p2[:, :DH], p2[:, DH:]], axis=0)


def _tc3_body(hist_ref, s_ref, p_ref, W_ref, b_ref, out_ref):
    dinv = _dinv_of(hist_ref[...])
    sv = s_ref[...]
    pv = p_ref[...]
    t = jnp.concatenate([sv[0] + pv[0], sv[1] + pv[1]], axis=1) * dinv[:, None]
    out_ref[...] = jnp.dot(t, W_ref[...], preferred_element_type=jnp.float32) + b_ref[...]


_hist_spec = pl.BlockSpec((NC * NS, R), lambda i: (0, i))
_row_spec = pl.BlockSpec((R, D), lambda i: (i, 0))
_half_spec = pl.BlockSpec((2, R, DH), lambda i: (0, i, 0))
_vec_spec = pl.BlockSpec((1, D), lambda i: (0, 0))
_mat_spec = pl.BlockSpec((D, D), lambda i: (0, 0))

_p_shape = jax.ShapeDtypeStruct((2, N_PAD, DH), jnp.float32)

_tc1 = pl.pallas_call(
    _tc1_body, grid=(GRID,),
    in_specs=[_hist_spec, _row_spec, _vec_spec, _vec_spec],
    out_specs=_half_spec, out_shape=_p_shape)

_tc2 = pl.pallas_call(
    _tc2_body, grid=(GRID,),
    in_specs=[_hist_spec, _half_spec, _half_spec, _mat_spec, _vec_spec,
              _vec_spec, _vec_spec],
    out_specs=_half_spec, out_shape=_p_shape)

_tc3 = pl.pallas_call(
    _tc3_body, grid=(GRID,),
    in_specs=[_hist_spec, _half_spec, _half_spec, _mat_spec, _vec_spec],
    out_specs=_row_spec,
    out_shape=jax.ShapeDtypeStruct((N_PAD, D), jnp.float32))


@jax.jit
def _run(x, edge_index, ln1_gamma, ln1_beta, W1, b1, ln2_gamma, ln2_beta, W2, b2):
    src = edge_index[0].astype(jnp.int32)
    dst = edge_index[1].astype(jnp.int32)
    pad = E_PAD - E
    pad_src = jnp.arange(pad, dtype=jnp.int32) % N
    pad_dst = N + jnp.arange(pad, dtype=jnp.int32) % (N_PAD - N)
    src_p = jnp.concatenate([src, pad_src]).reshape(NCHUNKS, CHUNK)
    dst_p = jnp.concatenate([dst, pad_dst]).reshape(NCHUNKS, CHUNK)
    src3 = jnp.stack([src_p, src_p + N_PAD], axis=0)

    x_p = jnp.pad(x, ((0, N_PAD - N), (0, 0)))
    g1 = ln1_gamma.reshape(1, D)
    be1 = ln1_beta.reshape(1, D)
    g2 = ln2_gamma.reshape(1, D)
    be2 = ln2_beta.reshape(1, D)
    b1r = b1.reshape(1, D)
    b2r = b2.reshape(1, D)

    hist = _sc_degree(dst_p)

    p1 = _tc1(hist, x_p, g1, be1)
    s1 = _sc_propagate(p1.reshape(NC * N_PAD, DH), src3, dst_p)
    p2 = _tc2(hist, s1.reshape(2, N_PAD, DH), p1, W1, b1r, g2, be2)
    s2 = _sc_propagate(p2.reshape(NC * N_PAD, DH), src3, dst_p)
    out = _tc3(hist, s2.reshape(2, N_PAD, DH), p2, W2, b2r)
    return out[:N]


def kernel(x, edge_index, ln1_gamma, ln1_beta, W1, b1, ln2_gamma, ln2_beta, W2, b2):
    return _run(x, edge_index, ln1_gamma, ln1_beta, W1, b1,
                ln2_gamma, ln2_beta, W2, b2)


# speedup vs baseline: 2.3459x; 1.0557x over previous
import functools

import jax
import jax.numpy as jnp
from jax import lax
from jax.experimental import pallas as pl
from jax.experimental.pallas import tpu as pltpu
from jax.experimental.pallas import tpu_sc as plsc

N = 10000
E = 160000
D = 256
DH = D // 2

NC = 2
NS = 16
CHUNK = 128

N_PAD = 10240
ROWS_PER_TILE = N_PAD // NS
ROW_CHUNKS = ROWS_PER_TILE // CHUNK

E_PAD = 163840
NCHUNKS = E_PAD // CHUNK
CHUNKS_PER_TILE = NCHUNKS // NS
DEG_CHUNKS_PER_TILE = NCHUNKS // (NC * NS)

R = 1024
GRID = N_PAD // R

_mesh = plsc.VectorSubcoreMesh(core_axis_name="c", subcore_axis_name="s")


@functools.partial(
    pl.kernel,
    out_type=jax.ShapeDtypeStruct((NC * NS, N_PAD), jnp.float32),
    mesh=_mesh,
    scratch_types=[
        pltpu.VMEM((DEG_CHUNKS_PER_TILE, CHUNK), jnp.int32),
        pltpu.VMEM((N_PAD,), jnp.float32),
    ],
    compiler_params=pltpu.CompilerParams(needs_layout_passes=False),
)
def _sc_degree(dst_hbm, out_hbm, idx_v, hist_v):
    c = lax.axis_index("c")
    s = lax.axis_index("s")
    wid = c * NS + s

    pltpu.sync_copy(dst_hbm.at[pl.ds(wid * DEG_CHUNKS_PER_TILE, DEG_CHUNKS_PER_TILE)], idx_v)

    zeros16 = jnp.zeros((16,), jnp.float32)

    def zbody(i, carry):
        for v in range(8):
            hist_v[pl.ds((i * 8 + v) * 16, 16)] = zeros16
        return carry

    lax.fori_loop(0, N_PAD // 128, zbody, 0)

    ones16 = jnp.full((16,), 1.0, jnp.float32)

    def body(j, carry):
        for v in range(8):
            idx = idx_v[j, pl.ds(v * 16, 16)]
            plsc.addupdate_scatter(hist_v, [idx], ones16)
        return carry

    lax.fori_loop(0, DEG_CHUNKS_PER_TILE, body, 0)

    pltpu.sync_copy(hist_v, out_hbm.at[wid])


GC = 40
GROUPS = CHUNKS_PER_TILE // GC
GPAIRS = GC // 2


@functools.partial(
    pl.kernel,
    out_type=jax.ShapeDtypeStruct((NC * N_PAD, DH), jnp.float32),
    mesh=_mesh,
    scratch_types=[
        pltpu.VMEM((GC, CHUNK), jnp.int32),
        pltpu.VMEM((GC, CHUNK), jnp.int32),
        pltpu.VMEM((2, CHUNK, DH), jnp.float32),
        pltpu.VMEM_SHARED((N_PAD, DH), jnp.float32),
        pltpu.SemaphoreType.DMA,
        pltpu.SemaphoreType.DMA,
    ],
    compiler_params=pltpu.CompilerParams(needs_layout_passes=False),
)
def _sc_propagate(p_hbm, src_hbm, dst_hbm, out_hbm,
                  src_v, dst_v, gbuf, acc, sem0, sem1):
    c = lax.axis_index("c")
    s = lax.axis_index("s")
    base_chunk = s * CHUNKS_PER_TILE

    zeros16 = jnp.zeros((16,), jnp.float32)

    def zbody(i, carry):
        for v in range(8):
            gbuf[0, i, pl.ds(v * 16, 16)] = zeros16
        return carry

    lax.fori_loop(0, CHUNK, zbody, 0)
    for k in range(ROW_CHUNKS):
        pltpu.sync_copy(gbuf.at[0], acc.at[pl.ds(s * ROWS_PER_TILE + k * CHUNK, CHUNK)])
    plsc.subcore_barrier()

    for g in range(GROUPS):
        pltpu.sync_copy(src_hbm.at[c, pl.ds(base_chunk + g * GC, GC)], src_v)
        pltpu.sync_copy(dst_hbm.at[pl.ds(base_chunk + g * GC, GC)], dst_v)
        pltpu.async_copy(p_hbm.at[src_v.at[0]], gbuf.at[0], sem0)

        def body(jj, carry):
            j0 = 2 * jj
            j1 = j0 + 1
            pltpu.async_copy(p_hbm.at[src_v.at[j1]], gbuf.at[1], sem1)
            pltpu.make_async_copy(p_hbm.at[src_v.at[j0]], gbuf.at[0], sem0).wait()
            pltpu.sync_copy(gbuf.at[0], acc.at[dst_v.at[j0]], add=True)

            @pl.when(jj < GPAIRS - 1)
            def _():
                pltpu.async_copy(p_hbm.at[src_v.at[j0 + 2]], gbuf.at[0], sem0)

            pltpu.make_async_copy(p_hbm.at[src_v.at[j1]], gbuf.at[1], sem1).wait()
            pltpu.sync_copy(gbuf.at[1], acc.at[dst_v.at[j1]], add=True)
            return carry

        lax.fori_loop(0, GPAIRS, body, 0)
    plsc.subcore_barrier()

    for k in range(ROW_CHUNKS):
        rows = s * ROWS_PER_TILE + k * CHUNK
        pltpu.sync_copy(acc.at[pl.ds(rows, CHUNK)], gbuf.at[0])
        pltpu.sync_copy(gbuf.at[0], out_hbm.at[pl.ds(c * N_PAD + rows, CHUNK)])


def _dinv_of(hist_blk):
    deg = jnp.sum(hist_blk, axis=0) + 1.0
    return 1.0 / jnp.sqrt(deg)


def _ln(z, g, b):
    mu = jnp.mean(z, axis=-1, keepdims=True)
    var = jnp.mean((z - mu) ** 2, axis=-1, keepdims=True)
    return (z - mu) / jnp.sqrt(var + 1e-5) * g + b


def _tc1_body(hist_ref, x_ref, g_ref, b_ref, out_ref):
    dinv = _dinv_of(hist_ref[...])
    p = _ln(x_ref[...], g_ref[...], b_ref[...]) * dinv[:, None]
    out_ref[...] = jnp.stack([p[:, :DH], p[:, DH:]], axis=0)


def _tc2_body(hist_ref, s_ref, p_ref, W_ref, b_ref, g2_ref, be2_ref, out_ref):
    dinv = _dinv_of(hist_ref[...])
    sv = s_ref[...]
    pv = p_ref[...]
    t = jnp.concatenate([sv[0] + pv[0], sv[1] + pv[1]], axis=1) * dinv[:, None]
    h = jnp.dot(t, W_ref[...], preferred_element_type=jnp.float32) + b_ref[...]
    p2 = _ln(h, g2_ref[...], be2_ref[...]) * dinv[:, None]
    out_ref[...] = jnp.stack([p2[:, :DH], p2[:, DH:]], axis=0)


def _tc3_body(hist_ref, s_ref, p_ref, W_ref, b_ref, out_ref):
    dinv = _dinv_of(hist_ref[...])
    sv = s_ref[...]
    pv = p_ref[...]
    t = jnp.concatenate([sv[0] + pv[0], sv[1] + pv[1]], axis=1) * dinv[:, None]
    out_ref[...] = jnp.dot(t, W_ref[...], preferred_element_type=jnp.float32) + b_ref[...]


_hist_spec = pl.BlockSpec((NC * NS, R), lambda i: (0, i))
_row_spec = pl.BlockSpec((R, D), lambda i: (i, 0))
_half_spec = pl.BlockSpec((2, R, DH), lambda i: (0, i, 0))
_vec_spec = pl.BlockSpec((1, D), lambda i: (0, 0))
_mat_spec = pl.BlockSpec((D, D), lambda i: (0, 0))

_p_shape = jax.ShapeDtypeStruct((2, N_PAD, DH), jnp.float32)

_tc1 = pl.pallas_call(
    _tc1_body, grid=(GRID,),
    in_specs=[_hist_spec, _row_spec, _vec_spec, _vec_spec],
    out_specs=_half_spec, out_shape=_p_shape)

_tc2 = pl.pallas_call(
    _tc2_body, grid=(GRID,),
    in_specs=[_hist_spec, _half_spec, _half_spec, _mat_spec, _vec_spec,
              _vec_spec, _vec_spec],
    out_specs=_half_spec, out_shape=_p_shape)

_tc3 = pl.pallas_call(
    _tc3_body, grid=(GRID,),
    in_specs=[_hist_spec, _half_spec, _half_spec, _mat_spec, _vec_spec],
    out_specs=_row_spec,
    out_shape=jax.ShapeDtypeStruct((N_PAD, D), jnp.float32))


@jax.jit
def _run(x, edge_index, ln1_gamma, ln1_beta, W1, b1, ln2_gamma, ln2_beta, W2, b2):
    src = edge_index[0].astype(jnp.int32)
    dst = edge_index[1].astype(jnp.int32)
    pad = E_PAD - E
    pad_src = jnp.arange(pad, dtype=jnp.int32) % N
    pad_dst = N + jnp.arange(pad, dtype=jnp.int32) % (N_PAD - N)
    src_p = jnp.concatenate([src, pad_src]).reshape(NCHUNKS, CHUNK)
    dst_p = jnp.concatenate([dst, pad_dst]).reshape(NCHUNKS, CHUNK)
    src3 = jnp.stack([src_p, src_p + N_PAD], axis=0)

    x_p = jnp.pad(x, ((0, N_PAD - N), (0, 0)))
    g1 = ln1_gamma.reshape(1, D)
    be1 = ln1_beta.reshape(1, D)
    g2 = ln2_gamma.reshape(1, D)
    be2 = ln2_beta.reshape(1, D)
    b1r = b1.reshape(1, D)
    b2r = b2.reshape(1, D)

    hist = _sc_degree(dst_p)

    p1 = _tc1(hist, x_p, g1, be1)
    s1 = _sc_propagate(p1.reshape(NC * N_PAD, DH), src3, dst_p)
    p2 = _tc2(hist, s1.reshape(2, N_PAD, DH), p1, W1, b1r, g2, be2)
    s2 = _sc_propagate(p2.reshape(NC * N_PAD, DH), src3, dst_p)
    out = _tc3(hist, s2.reshape(2, N_PAD, DH), p2, W2, b2r)
    return out[:N]


def kernel(x, edge_index, ln1_gamma, ln1_beta, W1, b1, ln2_gamma, ln2_beta, W2, b2):
    return _run(x, edge_index, ln1_gamma, ln1_beta, W1, b1,
                ln2_gamma, ln2_beta, W2, b2)


# speedup vs baseline: 2.3561x; 1.0043x over previous
import functools

import jax
import jax.numpy as jnp
from jax import lax
from jax.experimental import pallas as pl
from jax.experimental.pallas import tpu as pltpu
from jax.experimental.pallas import tpu_sc as plsc

N = 10000
E = 160000
D = 256
DH = D // 2

NC = 2
NS = 16
CHUNK = 128

N_PAD = 10240
ROWS_PER_TILE = N_PAD // NS
ROW_CHUNKS = ROWS_PER_TILE // CHUNK

E_PAD = 163840
NCHUNKS = E_PAD // CHUNK
CHUNKS_PER_TILE = NCHUNKS // NS
DEG_CHUNKS_PER_TILE = NCHUNKS // (NC * NS)

R = 1024
GRID = N_PAD // R

_mesh = plsc.VectorSubcoreMesh(core_axis_name="c", subcore_axis_name="s")


@functools.partial(
    pl.kernel,
    out_type=jax.ShapeDtypeStruct((NC * NS, N_PAD), jnp.float32),
    mesh=_mesh,
    scratch_types=[
        pltpu.VMEM((DEG_CHUNKS_PER_TILE, CHUNK), jnp.int32),
        pltpu.VMEM((N_PAD,), jnp.float32),
    ],
    compiler_params=pltpu.CompilerParams(needs_layout_passes=False),
)
def _sc_degree(dst_hbm, out_hbm, idx_v, hist_v):
    c = lax.axis_index("c")
    s = lax.axis_index("s")
    wid = c * NS + s

    pltpu.sync_copy(dst_hbm.at[pl.ds(wid * DEG_CHUNKS_PER_TILE, DEG_CHUNKS_PER_TILE)], idx_v)

    zeros16 = jnp.zeros((16,), jnp.float32)

    def zbody(i, carry):
        for v in range(8):
            hist_v[pl.ds((i * 8 + v) * 16, 16)] = zeros16
        return carry

    lax.fori_loop(0, N_PAD // 128, zbody, 0)

    ones16 = jnp.full((16,), 1.0, jnp.float32)

    def body(j, carry):
        for v in range(8):
            idx = idx_v[j, pl.ds(v * 16, 16)]
            plsc.addupdate_scatter(hist_v, [idx], ones16)
        return carry

    lax.fori_loop(0, DEG_CHUNKS_PER_TILE, body, 0)

    pltpu.sync_copy(hist_v, out_hbm.at[wid])


GC = 40
GROUPS = CHUNKS_PER_TILE // GC
GPAIRS = GC // 2


@functools.partial(
    pl.kernel,
    out_type=jax.ShapeDtypeStruct((NC * N_PAD, DH), jnp.float32),
    mesh=_mesh,
    scratch_types=[
        pltpu.VMEM((GC, CHUNK), jnp.int32),
        pltpu.VMEM((GC, CHUNK), jnp.int32),
        pltpu.VMEM((2, CHUNK, DH), jnp.float32),
        pltpu.VMEM_SHARED((N_PAD, DH), jnp.float32),
        pltpu.SemaphoreType.DMA,
        pltpu.SemaphoreType.DMA,
    ],
    compiler_params=pltpu.CompilerParams(needs_layout_passes=False),
)
def _sc_propagate(p_hbm, src_hbm, dst_hbm, out_hbm,
                  src_v, dst_v, gbuf, acc, sem0, sem1):
    c = lax.axis_index("c")
    s = lax.axis_index("s")
    base_chunk = s * CHUNKS_PER_TILE

    zeros16 = jnp.zeros((16,), jnp.float32)

    def zbody(i, carry):
        for v in range(8):
            gbuf[0, i, pl.ds(v * 16, 16)] = zeros16
        return carry

    lax.fori_loop(0, CHUNK, zbody, 0)
    for k in range(ROW_CHUNKS):
        pltpu.sync_copy(gbuf.at[0], acc.at[pl.ds(s * ROWS_PER_TILE + k * CHUNK, CHUNK)])
    plsc.subcore_barrier()

    for g in range(GROUPS):
        pltpu.sync_copy(src_hbm.at[c, pl.ds(base_chunk + g * GC, GC)], src_v)
        pltpu.sync_copy(dst_hbm.at[pl.ds(base_chunk + g * GC, GC)], dst_v)
        pltpu.async_copy(p_hbm.at[src_v.at[0]], gbuf.at[0], sem0)

        def body(jj, carry):
            j0 = 2 * jj
            j1 = j0 + 1
            pltpu.async_copy(p_hbm.at[src_v.at[j1]], gbuf.at[1], sem1)
            pltpu.make_async_copy(p_hbm.at[src_v.at[j0]], gbuf.at[0], sem0).wait()
            pltpu.sync_copy(gbuf.at[0], acc.at[dst_v.at[j0]], add=True)

            @pl.when(jj < GPAIRS - 1)
            def _():
                pltpu.async_copy(p_hbm.at[src_v.at[j0 + 2]], gbuf.at[0], sem0)

            pltpu.make_async_copy(p_hbm.at[src_v.at[j1]], gbuf.at[1], sem1).wait()
            pltpu.sync_copy(gbuf.at[1], acc.at[dst_v.at[j1]], add=True)
            return carry

        lax.fori_loop(0, GPAIRS, body, 0)
    plsc.subcore_barrier()

    for k in range(ROW_CHUNKS):
        rows = s * ROWS_PER_TILE + k * CHUNK
        pltpu.sync_copy(acc.at[pl.ds(rows, CHUNK)], gbuf.at[0])
        pltpu.sync_copy(gbuf.at[0], out_hbm.at[pl.ds(c * N_PAD + rows, CHUNK)])


def _dinv_of(hist_blk):
    deg = jnp.sum(hist_blk, axis=0) + 1.0
    return 1.0 / jnp.sqrt(deg)


def _ln(z, g, b):
    mu = jnp.mean(z, axis=-1, keepdims=True)
    var = jnp.mean((z - mu) ** 2, axis=-1, keepdims=True)
    return (z - mu) / jnp.sqrt(var + 1e-5) * g + b


def _tc1_body(hist_ref, x_ref, g_ref, b_ref, out_ref, dinv_ref):
    dinv = _dinv_of(hist_ref[...])
    dinv_ref[...] = dinv[None, :]
    p = _ln(x_ref[...], g_ref[...], b_ref[...]) * dinv[:, None]
    out_ref[...] = jnp.stack([p[:, :DH], p[:, DH:]], axis=0)


def _tc2_body(dinv_ref, s_ref, p_ref, W_ref, b_ref, g2_ref, be2_ref, out_ref):
    dinv = dinv_ref[0, :]
    sv = s_ref[...]
    pv = p_ref[...]
    t = jnp.concatenate([sv[0] + pv[0], sv[1] + pv[1]], axis=1) * dinv[:, None]
    h = jnp.dot(t, W_ref[...], preferred_element_type=jnp.float32) + b_ref[...]
    p2 = _ln(h, g2_ref[...], be2_ref[...]) * dinv[:, None]
    out_ref[...] = jnp.stack([p2[:, :DH], p2[:, DH:]], axis=0)


def _tc3_body(dinv_ref, s_ref, p_ref, W_ref, b_ref, out_ref):
    dinv = dinv_ref[0, :]
    sv = s_ref[...]
    pv = p_ref[...]
    t = jnp.concatenate([sv[0] + pv[0], sv[1] + pv[1]], axis=1) * dinv[:, None]
    out_ref[...] = jnp.dot(t, W_ref[...], preferred_element_type=jnp.float32) + b_ref[...]


_hist_spec = pl.BlockSpec((NC * NS, R), lambda i: (0, i))
_dinv_spec = pl.BlockSpec((1, R), lambda i: (0, i))
_row_spec = pl.BlockSpec((R, D), lambda i: (i, 0))
_half_spec = pl.BlockSpec((2, R, DH), lambda i: (0, i, 0))
_vec_spec = pl.BlockSpec((1, D), lambda i: (0, 0))
_mat_spec = pl.BlockSpec((D, D), lambda i: (0, 0))

_p_shape = jax.ShapeDtypeStruct((2, N_PAD, DH), jnp.float32)

_tc1 = pl.pallas_call(
    _tc1_body, grid=(GRID,),
    in_specs=[_hist_spec, _row_spec, _vec_spec, _vec_spec],
    out_specs=(_half_spec, _dinv_spec),
    out_shape=(_p_shape, jax.ShapeDtypeStruct((1, N_PAD), jnp.float32)))

_tc2 = pl.pallas_call(
    _tc2_body, grid=(GRID,),
    in_specs=[_dinv_spec, _half_spec, _half_spec, _mat_spec, _vec_spec,
              _vec_spec, _vec_spec],
    out_specs=_half_spec, out_shape=_p_shape)

_tc3 = pl.pallas_call(
    _tc3_body, grid=(GRID,),
    in_specs=[_dinv_spec, _half_spec, _half_spec, _mat_spec, _vec_spec],
    out_specs=_row_spec,
    out_shape=jax.ShapeDtypeStruct((N_PAD, D), jnp.float32))


@jax.jit
def _run(x, edge_index, ln1_gamma, ln1_beta, W1, b1, ln2_gamma, ln2_beta, W2, b2):
    src = edge_index[0].astype(jnp.int32)
    dst = edge_index[1].astype(jnp.int32)
    pad = E_PAD - E
    pad_src = jnp.arange(pad, dtype=jnp.int32) % N
    pad_dst = N + jnp.arange(pad, dtype=jnp.int32) % (N_PAD - N)
    src_p = jnp.concatenate([src, pad_src]).reshape(NCHUNKS, CHUNK)
    dst_p = jnp.concatenate([dst, pad_dst]).reshape(NCHUNKS, CHUNK)
    src3 = jnp.stack([src_p, src_p + N_PAD], axis=0)

    x_p = jnp.pad(x, ((0, N_PAD - N), (0, 0)))
    g1 = ln1_gamma.reshape(1, D)
    be1 = ln1_beta.reshape(1, D)
    g2 = ln2_gamma.reshape(1, D)
    be2 = ln2_beta.reshape(1, D)
    b1r = b1.reshape(1, D)
    b2r = b2.reshape(1, D)

    hist = _sc_degree(dst_p)

    p1, dinv = _tc1(hist, x_p, g1, be1)
    s1 = _sc_propagate(p1.reshape(NC * N_PAD, DH), src3, dst_p)
    p2 = _tc2(dinv, s1.reshape(2, N_PAD, DH), p1, W1, b1r, g2, be2)
    s2 = _sc_propagate(p2.reshape(NC * N_PAD, DH), src3, dst_p)
    out = _tc3(dinv, s2.reshape(2, N_PAD, DH), p2, W2, b2r)
    return out[:N]


def kernel(x, edge_index, ln1_gamma, ln1_beta, W1, b1, ln2_gamma, ln2_beta, W2, b2):
    return _run(x, edge_index, ln1_gamma, ln1_beta, W1, b1,
                ln2_gamma, ln2_beta, W2, b2)


# speedup vs baseline: 2.3832x; 1.0115x over previous
import functools

import jax
import jax.numpy as jnp
from jax import lax
from jax.experimental import pallas as pl
from jax.experimental.pallas import tpu as pltpu
from jax.experimental.pallas import tpu_sc as plsc

N = 10000
E = 160000
D = 256
DH = D // 2

NC = 2
NS = 16
CHUNK = 128

N_PAD = 10240
ROWS_PER_TILE = N_PAD // NS
ROW_CHUNKS = ROWS_PER_TILE // CHUNK

E_PAD = 163840
NCHUNKS = E_PAD // CHUNK
CHUNKS_PER_TILE = NCHUNKS // NS
DEG_CHUNKS_PER_TILE = NCHUNKS // (NC * NS)

R = 1024
GRID = N_PAD // R

_mesh = plsc.VectorSubcoreMesh(core_axis_name="c", subcore_axis_name="s")


@functools.partial(
    pl.kernel,
    out_type=jax.ShapeDtypeStruct((NC * NS, N_PAD), jnp.float32),
    mesh=_mesh,
    scratch_types=[
        pltpu.VMEM((DEG_CHUNKS_PER_TILE, CHUNK), jnp.int32),
        pltpu.VMEM((N_PAD,), jnp.float32),
    ],
    compiler_params=pltpu.CompilerParams(needs_layout_passes=False),
)
def _sc_degree(dst_hbm, out_hbm, idx_v, hist_v):
    c = lax.axis_index("c")
    s = lax.axis_index("s")
    wid = c * NS + s

    pltpu.sync_copy(dst_hbm.at[pl.ds(wid * DEG_CHUNKS_PER_TILE, DEG_CHUNKS_PER_TILE)], idx_v)

    zeros16 = jnp.zeros((16,), jnp.float32)

    def zbody(i, carry):
        for v in range(8):
            hist_v[pl.ds((i * 8 + v) * 16, 16)] = zeros16
        return carry

    lax.fori_loop(0, N_PAD // 128, zbody, 0)

    ones16 = jnp.full((16,), 1.0, jnp.float32)

    def body(j, carry):
        for v in range(8):
            idx = idx_v[j, pl.ds(v * 16, 16)]
            plsc.addupdate_scatter(hist_v, [idx], ones16)
        return carry

    lax.fori_loop(0, DEG_CHUNKS_PER_TILE, body, 0)

    pltpu.sync_copy(hist_v, out_hbm.at[wid])


GC = 40
GROUPS = CHUNKS_PER_TILE // GC
GPAIRS = GC // 2


@functools.partial(
    pl.kernel,
    out_type=jax.ShapeDtypeStruct((NC * N_PAD, DH), jnp.float32),
    mesh=_mesh,
    scratch_types=[
        pltpu.VMEM((GC, CHUNK), jnp.int32),
        pltpu.VMEM((GC, CHUNK), jnp.int32),
        pltpu.VMEM((2, CHUNK, DH), jnp.float32),
        pltpu.VMEM_SHARED((N_PAD, DH), jnp.float32),
        pltpu.SemaphoreType.DMA,
        pltpu.SemaphoreType.DMA,
    ],
    compiler_params=pltpu.CompilerParams(needs_layout_passes=False),
)
def _sc_propagate(p_hbm, src_hbm, dst_hbm, out_hbm,
                  src_v, dst_v, gbuf, acc, sem0, sem1):
    c = lax.axis_index("c")
    s = lax.axis_index("s")
    base_chunk = s * CHUNKS_PER_TILE

    zeros16 = jnp.zeros((16,), jnp.float32)

    def zbody(i, carry):
        for v in range(8):
            gbuf[0, i, pl.ds(v * 16, 16)] = zeros16
        return carry

    lax.fori_loop(0, CHUNK, zbody, 0)
    for k in range(ROW_CHUNKS):
        pltpu.sync_copy(gbuf.at[0], acc.at[pl.ds(s * ROWS_PER_TILE + k * CHUNK, CHUNK)])
    plsc.subcore_barrier()

    for g in range(GROUPS):
        pltpu.sync_copy(src_hbm.at[c, pl.ds(base_chunk + g * GC, GC)], src_v)
        pltpu.sync_copy(dst_hbm.at[pl.ds(base_chunk + g * GC, GC)], dst_v)
        pltpu.async_copy(p_hbm.at[src_v.at[0]], gbuf.at[0], sem0)

        def body(jj, carry):
            j0 = 2 * jj
            j1 = j0 + 1
            pltpu.async_copy(p_hbm.at[src_v.at[j1]], gbuf.at[1], sem1)
            pltpu.make_async_copy(p_hbm.at[src_v.at[j0]], gbuf.at[0], sem0).wait()
            pltpu.sync_copy(gbuf.at[0], acc.at[dst_v.at[j0]], add=True)

            @pl.when(jj < GPAIRS - 1)
            def _():
                pltpu.async_copy(p_hbm.at[src_v.at[j0 + 2]], gbuf.at[0], sem0)

            pltpu.make_async_copy(p_hbm.at[src_v.at[j1]], gbuf.at[1], sem1).wait()
            pltpu.sync_copy(gbuf.at[1], acc.at[dst_v.at[j1]], add=True)
            return carry

        lax.fori_loop(0, GPAIRS, body, 0)
    plsc.subcore_barrier()

    sems = (sem0, sem1)
    for k in range(ROW_CHUNKS):
        rows = s * ROWS_PER_TILE + k * CHUNK
        b = k % 2
        if k >= 2:
            prows = s * ROWS_PER_TILE + (k - 2) * CHUNK
            pltpu.make_async_copy(
                gbuf.at[b], out_hbm.at[pl.ds(c * N_PAD + prows, CHUNK)], sems[b]).wait()
        pltpu.sync_copy(acc.at[pl.ds(rows, CHUNK)], gbuf.at[b])
        pltpu.async_copy(gbuf.at[b], out_hbm.at[pl.ds(c * N_PAD + rows, CHUNK)], sems[b])
    for k in range(ROW_CHUNKS - 2, ROW_CHUNKS):
        rows = s * ROWS_PER_TILE + k * CHUNK
        pltpu.make_async_copy(
            gbuf.at[k % 2], out_hbm.at[pl.ds(c * N_PAD + rows, CHUNK)], sems[k % 2]).wait()


def _dinv_of(hist_blk):
    deg = jnp.sum(hist_blk, axis=0) + 1.0
    return 1.0 / jnp.sqrt(deg)


def _ln(z, g, b):
    mu = jnp.mean(z, axis=-1, keepdims=True)
    var = jnp.mean((z - mu) ** 2, axis=-1, keepdims=True)
    return (z - mu) / jnp.sqrt(var + 1e-5) * g + b


def _tc1_body(hist_ref, x_ref, g_ref, b_ref, out_ref, dinv_ref):
    dinv = _dinv_of(hist_ref[...])
    dinv_ref[...] = dinv[None, :]
    p = _ln(x_ref[...], g_ref[...], b_ref[...]) * dinv[:, None]
    out_ref[...] = jnp.stack([p[:, :DH], p[:, DH:]], axis=0)


def _tc2_body(dinv_ref, s_ref, p_ref, W_ref, b_ref, g2_ref, be2_ref, out_ref):
    dinv = dinv_ref[0, :]
    sv = s_ref[...]
    pv = p_ref[...]
    t = jnp.concatenate([sv[0] + pv[0], sv[1] + pv[1]], axis=1) * dinv[:, None]
    h = jnp.dot(t, W_ref[...], preferred_element_type=jnp.float32) + b_ref[...]
    p2 = _ln(h, g2_ref[...], be2_ref[...]) * dinv[:, None]
    out_ref[...] = jnp.stack([p2[:, :DH], p2[:, DH:]], axis=0)


def _tc3_body(dinv_ref, s_ref, p_ref, W_ref, b_ref, out_ref):
    dinv = dinv_ref[0, :]
    sv = s_ref[...]
    pv = p_ref[...]
    t = jnp.concatenate([sv[0] + pv[0], sv[1] + pv[1]], axis=1) * dinv[:, None]
    out_ref[...] = jnp.dot(t, W_ref[...], preferred_element_type=jnp.float32) + b_ref[...]


_hist_spec = pl.BlockSpec((NC * NS, R), lambda i: (0, i))
_dinv_spec = pl.BlockSpec((1, R), lambda i: (0, i))
_row_spec = pl.BlockSpec((R, D), lambda i: (i, 0))
_half_spec = pl.BlockSpec((2, R, DH), lambda i: (0, i, 0))
_vec_spec = pl.BlockSpec((1, D), lambda i: (0, 0))
_mat_spec = pl.BlockSpec((D, D), lambda i: (0, 0))

_p_shape = jax.ShapeDtypeStruct((2, N_PAD, DH), jnp.float32)

_tc1 = pl.pallas_call(
    _tc1_body, grid=(GRID,),
    in_specs=[_hist_spec, _row_spec, _vec_spec, _vec_spec],
    out_specs=(_half_spec, _dinv_spec),
    out_shape=(_p_shape, jax.ShapeDtypeStruct((1, N_PAD), jnp.float32)))

_tc2 = pl.pallas_call(
    _tc2_body, grid=(GRID,),
    in_specs=[_dinv_spec, _half_spec, _half_spec, _mat_spec, _vec_spec,
              _vec_spec, _vec_spec],
    out_specs=_half_spec, out_shape=_p_shape)

_tc3 = pl.pallas_call(
    _tc3_body, grid=(GRID,),
    in_specs=[_dinv_spec, _half_spec, _half_spec, _mat_spec, _vec_spec],
    out_specs=_row_spec,
    out_shape=jax.ShapeDtypeStruct((N_PAD, D), jnp.float32))


@jax.jit
def _run(x, edge_index, ln1_gamma, ln1_beta, W1, b1, ln2_gamma, ln2_beta, W2, b2):
    src = edge_index[0].astype(jnp.int32)
    dst = edge_index[1].astype(jnp.int32)
    pad = E_PAD - E
    pad_src = jnp.arange(pad, dtype=jnp.int32) % N
    pad_dst = N + jnp.arange(pad, dtype=jnp.int32) % (N_PAD - N)
    src_p = jnp.concatenate([src, pad_src]).reshape(NCHUNKS, CHUNK)
    dst_p = jnp.concatenate([dst, pad_dst]).reshape(NCHUNKS, CHUNK)
    src3 = jnp.stack([src_p, src_p + N_PAD], axis=0)

    x_p = jnp.pad(x, ((0, N_PAD - N), (0, 0)))
    g1 = ln1_gamma.reshape(1, D)
    be1 = ln1_beta.reshape(1, D)
    g2 = ln2_gamma.reshape(1, D)
    be2 = ln2_beta.reshape(1, D)
    b1r = b1.reshape(1, D)
    b2r = b2.reshape(1, D)

    hist = _sc_degree(dst_p)

    p1, dinv = _tc1(hist, x_p, g1, be1)
    s1 = _sc_propagate(p1.reshape(NC * N_PAD, DH), src3, dst_p)
    p2 = _tc2(dinv, s1.reshape(2, N_PAD, DH), p1, W1, b1r, g2, be2)
    s2 = _sc_propagate(p2.reshape(NC * N_PAD, DH), src3, dst_p)
    out = _tc3(dinv, s2.reshape(2, N_PAD, DH), p2, W2, b2r)
    return out[:N]


def kernel(x, edge_index, ln1_gamma, ln1_beta, W1, b1, ln2_gamma, ln2_beta, W2, b2):
    return _run(x, edge_index, ln1_gamma, ln1_beta, W1, b1,
                ln2_gamma, ln2_beta, W2, b2)
